# Initial kernel scaffold; baseline (speedup 1.0000x reference)
import jax
import jax.numpy as jnp
from jax.experimental import pallas as pl


def kernel(x, edge_index, W1_l, W1_r, b1, W2_l, W2_r, b2):
    raise NotImplementedError("write your pallas kernel here")


# speedup vs baseline: 2.4761x; 2.4761x over previous
import functools

import jax
import jax.numpy as jnp
from jax import lax
from jax.experimental import pallas as pl
from jax.experimental.pallas import tpu as pltpu
from jax.experimental.pallas import tpu_sc as plsc

N_NODES = 100000
N_EDGES = 1600000

NC, NS = 2, 16
EPAD = 1638400
EGROUPS = EPAD // 128
GPS = EGROUPS // NS
NP = 100352


def _make_agg(F, R, passes_per_sc, KC):
    P = NC * passes_per_sc
    ACC_ROWS = R + 128
    PER_SUB = ACC_ROWS // NS
    OUT_PER_SUB = R // NS
    assert ACC_ROWS % NS == 0 and R % NS == 0 and GPS % KC == 0
    mesh = plsc.VectorSubcoreMesh(core_axis_name="c", subcore_axis_name="s")

    @functools.partial(
        pl.kernel,
        out_type=jax.ShapeDtypeStruct((P * R, F), jnp.float32),
        mesh=mesh,
        compiler_params=pltpu.CompilerParams(use_tc_tiling_on_sc=False),
        scratch_types=[
            pltpu.VMEM_SHARED((ACC_ROWS, F), jnp.float32),
            pltpu.VMEM((KC, 128), jnp.int32),
            pltpu.VMEM((KC, 128), jnp.int32),
            pltpu.VMEM((KC, 128), jnp.int32),
            pltpu.VMEM((128, F), jnp.float32),
            pltpu.VMEM((128, F), jnp.float32),
            pltpu.VMEM((128, F), jnp.float32),
            pltpu.SemaphoreType.DMA,
            pltpu.SemaphoreType.DMA,
        ],
    )
    def agg(tbl, srcg, dstg, out, acc, src_b, dst_b, loc_b, zrow, rows0, rows1,
            sem0, sem1):
        c = lax.axis_index("c")
        s = lax.axis_index("s")

        def _zr(j, _):
            for l in range(F // 16):
                zrow[j, pl.ds(16 * l, 16)] = jnp.zeros((16,), jnp.float32)
            return 0
        lax.fori_loop(0, 128, _zr, 0)

        for pp in range(passes_per_sc):
            p = c * passes_per_sc + pp
            lo = p * R

            z0 = s * PER_SUB
            nfull = PER_SUB // 128
            def _zero(i, _):
                pltpu.sync_copy(zrow, acc.at[pl.ds(z0 + i * 128, 128)])
                return 0
            lax.fori_loop(0, nfull, _zero, 0)
            rem = PER_SUB - nfull * 128
            if rem:
                pltpu.sync_copy(zrow.at[pl.ds(0, rem)],
                                acc.at[pl.ds(z0 + nfull * 128, rem)])
            plsc.subcore_barrier()

            def _chunk(k, _):
                row0 = s * GPS + k * KC
                pltpu.sync_copy(srcg.at[pl.ds(row0, KC)], src_b)
                pltpu.sync_copy(dstg.at[pl.ds(row0, KC)], dst_b)
                for j in range(KC):
                    for l in range(8):
                        d = dst_b[j, pl.ds(16 * l, 16)]
                        rel = d - lo
                        ok = (rel >= 0) & (rel < R)
                        loc_b[j, pl.ds(16 * l, 16)] = jnp.where(ok, rel, R)
                bufs = (rows0, rows1)
                sems = (sem0, sem1)
                cps = [None, None]
                cps[0] = pltpu.async_copy(tbl.at[src_b.at[0]], rows0, sem0)
                for j in range(KC):
                    nj = j + 1
                    if nj < KC:
                        cps[nj % 2] = pltpu.async_copy(
                            tbl.at[src_b.at[nj]], bufs[nj % 2], sems[nj % 2])
                    cps[j % 2].wait()
                    pltpu.sync_copy(bufs[j % 2], acc.at[loc_b.at[j]], add=True)
                return 0
            lax.fori_loop(0, GPS // KC, _chunk, 0)
            plsc.subcore_barrier()

            o0 = s * OUT_PER_SUB
            pltpu.sync_copy(acc.at[pl.ds(o0, OUT_PER_SUB)],
                            out.at[pl.ds(lo + o0, OUT_PER_SUB)])
            plsc.subcore_barrier()

    return agg


_agg32 = _make_agg(F=32, R=51200, passes_per_sc=1, KC=8)
_agg64 = _make_agg(F=64, R=25600, passes_per_sc=2, KC=8)


def _dense_mid(s1, xp, w1l, w1r, b1, w2l, w2r, b2):
    B = 512
    grid = (NP // B,)

    def body(s1_r, xp_r, w1l_r, w1r_r, b1_r, w2l_r, w2r_r, b2_r, g_r, r_r):
        s1b = s1_r[...]
        inv = 1.0 / jnp.maximum(s1b[:, 27:28], 1.0)
        mean = s1b * inv
        h = jnp.maximum(
            jnp.dot(mean, w1l_r[...], preferred_element_type=jnp.float32)
            + jnp.dot(xp_r[...], w1r_r[...], preferred_element_type=jnp.float32)
            + b1_r[...], 0.0)
        g_r[...] = jnp.dot(h, w2l_r[...], preferred_element_type=jnp.float32)
        r_r[...] = (jnp.dot(h, w2r_r[...], preferred_element_type=jnp.float32)
                    + b2_r[...])

    return pl.pallas_call(
        body,
        grid=grid,
        in_specs=[
            pl.BlockSpec((B, 32), lambda i: (i, 0)),
            pl.BlockSpec((B, 32), lambda i: (i, 0)),
            pl.BlockSpec((32, 128), lambda i: (0, 0)),
            pl.BlockSpec((32, 128), lambda i: (0, 0)),
            pl.BlockSpec((1, 128), lambda i: (0, 0)),
            pl.BlockSpec((128, 64), lambda i: (0, 0)),
            pl.BlockSpec((128, 64), lambda i: (0, 0)),
            pl.BlockSpec((1, 64), lambda i: (0, 0)),
        ],
        out_specs=[
            pl.BlockSpec((B, 64), lambda i: (i, 0)),
            pl.BlockSpec((B, 64), lambda i: (i, 0)),
        ],
        out_shape=[
            jax.ShapeDtypeStruct((NP, 64), jnp.float32),
            jax.ShapeDtypeStruct((NP, 64), jnp.float32),
        ],
    )(s1, xp, w1l, w1r, b1, w2l, w2r, b2)


def _final(s2, s1, r):
    B = 512
    grid = (NP // B,)

    def body(s2_r, s1_r, r_r, out_r):
        inv = 1.0 / jnp.maximum(s1_r[:, 27:28], 1.0)
        out_r[...] = s2_r[...] * inv + r_r[...]

    return pl.pallas_call(
        body,
        grid=grid,
        in_specs=[
            pl.BlockSpec((B, 64), lambda i: (i, 0)),
            pl.BlockSpec((B, 32), lambda i: (i, 0)),
            pl.BlockSpec((B, 64), lambda i: (i, 0)),
        ],
        out_specs=pl.BlockSpec((B, 64), lambda i: (i, 0)),
        out_shape=jax.ShapeDtypeStruct((NP, 64), jnp.float32),
    )(s2, s1, r)


def kernel(x, edge_index, W1_l, W1_r, b1, W2_l, W2_r, b2):
    src = edge_index[0].astype(jnp.int32)
    dst = edge_index[1].astype(jnp.int32)
    srcp = jnp.concatenate(
        [src, jnp.zeros((EPAD - N_EDGES,), jnp.int32)]).reshape(EGROUPS, 128)
    dstp = jnp.concatenate(
        [dst, jnp.full((EPAD - N_EDGES,), 1 << 29, jnp.int32)]).reshape(EGROUPS, 128)

    xp = jnp.concatenate(
        [x, jnp.ones((N_NODES, 1), jnp.float32),
         jnp.zeros((N_NODES, 4), jnp.float32)], axis=1)
    xp = jnp.concatenate(
        [xp, jnp.zeros((NP - N_NODES, 32), jnp.float32)], axis=0)

    S1 = _agg32(xp, srcp, dstp)
    w1l = jnp.pad(W1_l, ((0, 5), (0, 0)))
    w1r = jnp.pad(W1_r, ((0, 5), (0, 0)))
    g, r = _dense_mid(S1[:NP], xp, w1l, w1r,
                      b1.reshape(1, 128), W2_l, W2_r, b2.reshape(1, 64))
    S2 = _agg64(g, srcp, dstp)
    out = _final(S2[:NP], S1[:NP], r)
    return out[:N_NODES]


# speedup vs baseline: 2.4849x; 1.0035x over previous
import functools

import jax
import jax.numpy as jnp
from jax import lax
from jax.experimental import pallas as pl
from jax.experimental.pallas import tpu as pltpu
from jax.experimental.pallas import tpu_sc as plsc

N_NODES = 100000
N_EDGES = 1600000

NC, NS = 2, 16
EPAD = 1638400
EGROUPS = EPAD // 128
GPS = EGROUPS // NS
NP = 100352


def _make_agg(F, R, passes_per_sc, KC):
    P = NC * passes_per_sc
    ACC_ROWS = R + 128
    PER_SUB = ACC_ROWS // NS
    OUT_PER_SUB = R // NS
    assert ACC_ROWS % NS == 0 and R % NS == 0 and GPS % KC == 0
    mesh = plsc.VectorSubcoreMesh(core_axis_name="c", subcore_axis_name="s")

    @functools.partial(
        pl.kernel,
        out_type=jax.ShapeDtypeStruct((P * R, F), jnp.float32),
        mesh=mesh,
        compiler_params=pltpu.CompilerParams(use_tc_tiling_on_sc=False,
                                             needs_layout_passes=False),
        scratch_types=[
            pltpu.VMEM_SHARED((ACC_ROWS, F), jnp.float32),
            pltpu.VMEM((KC, 128), jnp.int32),
            pltpu.VMEM((KC, 128), jnp.int32),
            pltpu.VMEM((KC, 128), jnp.int32),
            pltpu.VMEM((KC, 128), jnp.int32),
            pltpu.VMEM((128, F), jnp.float32),
            pltpu.VMEM((128, F), jnp.float32),
            pltpu.VMEM((128, F), jnp.float32),
            pltpu.SemaphoreType.DMA,
            pltpu.SemaphoreType.DMA,
        ],
    )
    def agg(tbl, srcg, dstg, out, acc, src_b, dst_b, loc_b, fsrc_b, zrow,
            rows0, rows1, sem0, sem1):
        c = lax.axis_index("c")
        s = lax.axis_index("s")

        def _zr(j, _):
            for l in range(F // 16):
                zrow[j, pl.ds(16 * l, 16)] = jnp.zeros((16,), jnp.float32)
            return 0
        lax.fori_loop(0, 128, _zr, 0)

        for pp in range(passes_per_sc):
            p = c * passes_per_sc + pp
            lo = p * R

            z0 = s * PER_SUB
            nfull = PER_SUB // 128
            def _zero(i, _):
                pltpu.sync_copy(zrow, acc.at[pl.ds(z0 + i * 128, 128)])
                return 0
            lax.fori_loop(0, nfull, _zero, 0)
            rem = PER_SUB - nfull * 128
            if rem:
                pltpu.sync_copy(zrow.at[pl.ds(0, rem)],
                                acc.at[pl.ds(z0 + nfull * 128, rem)])
            plsc.subcore_barrier()

            def _chunk(k, _):
                row0 = s * GPS + k * KC
                pltpu.sync_copy(srcg.at[pl.ds(row0, KC)], src_b)
                pltpu.sync_copy(dstg.at[pl.ds(row0, KC)], dst_b)
                off = jnp.zeros((16,), jnp.int32)
                one = jnp.ones((16,), jnp.int32)
                zero = jnp.zeros((16,), jnp.int32)
                for j in range(KC):
                    for l in range(8):
                        d = dst_b[j, pl.ds(16 * l, 16)]
                        rel = d - lo
                        ok = (rel >= 0) & (rel < R)
                        sv = src_b[j, pl.ds(16 * l, 16)]
                        pos = off + plsc.cumsum(jnp.where(ok, one, zero)) - 1
                        prow = lax.shift_right_logical(pos, 7)
                        pcol = pos & 127
                        plsc.store_scatter(loc_b, [prow, pcol], rel, mask=ok)
                        plsc.store_scatter(fsrc_b, [prow, pcol], sv, mask=ok)
                        off = off + plsc.all_reduce_population_count(ok)
                n = jnp.max(off)
                ng = lax.shift_right_logical(n + 127, 7)
                total = lax.shift_left(ng, 7)
                it = jnp.arange(16, dtype=jnp.int32)
                for t in range(8):
                    pos = n + 16 * t + it
                    pm = pos < total
                    prow = lax.shift_right_logical(pos, 7)
                    pcol = pos & 127
                    plsc.store_scatter(loc_b, [prow, pcol], R + pcol, mask=pm)
                    plsc.store_scatter(fsrc_b, [prow, pcol], zero, mask=pm)
                bufs = (rows0, rows1)
                sems = (sem0, sem1)
                @pl.when(0 < ng)
                def _():
                    pltpu.async_copy(tbl.at[fsrc_b.at[0]], rows0, sem0)
                for g in range(KC):
                    if g + 1 < KC:
                        @pl.when(g + 1 < ng)
                        def _():
                            pltpu.async_copy(tbl.at[fsrc_b.at[g + 1]],
                                             bufs[(g + 1) % 2], sems[(g + 1) % 2])
                    @pl.when(g < ng)
                    def _():
                        pltpu.make_async_copy(tbl.at[fsrc_b.at[g]],
                                              bufs[g % 2], sems[g % 2]).wait()
                        pltpu.sync_copy(bufs[g % 2], acc.at[loc_b.at[g]],
                                        add=True)
                return 0
            lax.fori_loop(0, GPS // KC, _chunk, 0)
            plsc.subcore_barrier()

            o0 = s * OUT_PER_SUB
            pltpu.sync_copy(acc.at[pl.ds(o0, OUT_PER_SUB)],
                            out.at[pl.ds(lo + o0, OUT_PER_SUB)])
            plsc.subcore_barrier()

    return agg


_agg32 = _make_agg(F=32, R=25600, passes_per_sc=2, KC=16)
_agg64 = _make_agg(F=64, R=16768, passes_per_sc=3, KC=16)


def _dense_mid(s1, xp, w1l, w1r, b1, w2l, w2r, b2):
    B = 512
    grid = (NP // B,)

    def body(s1_r, xp_r, w1l_r, w1r_r, b1_r, w2l_r, w2r_r, b2_r, g_r, r_r):
        s1b = s1_r[...]
        inv = 1.0 / jnp.maximum(s1b[:, 27:28], 1.0)
        mean = s1b * inv
        h = jnp.maximum(
            jnp.dot(mean, w1l_r[...], preferred_element_type=jnp.float32)
            + jnp.dot(xp_r[...], w1r_r[...], preferred_element_type=jnp.float32)
            + b1_r[...], 0.0)
        g_r[...] = jnp.dot(h, w2l_r[...], preferred_element_type=jnp.float32)
        r_r[...] = (jnp.dot(h, w2r_r[...], preferred_element_type=jnp.float32)
                    + b2_r[...])

    return pl.pallas_call(
        body,
        grid=grid,
        in_specs=[
            pl.BlockSpec((B, 32), lambda i: (i, 0)),
            pl.BlockSpec((B, 32), lambda i: (i, 0)),
            pl.BlockSpec((32, 128), lambda i: (0, 0)),
            pl.BlockSpec((32, 128), lambda i: (0, 0)),
            pl.BlockSpec((1, 128), lambda i: (0, 0)),
            pl.BlockSpec((128, 64), lambda i: (0, 0)),
            pl.BlockSpec((128, 64), lambda i: (0, 0)),
            pl.BlockSpec((1, 64), lambda i: (0, 0)),
        ],
        out_specs=[
            pl.BlockSpec((B, 64), lambda i: (i, 0)),
            pl.BlockSpec((B, 64), lambda i: (i, 0)),
        ],
        out_shape=[
            jax.ShapeDtypeStruct((NP, 64), jnp.float32),
            jax.ShapeDtypeStruct((NP, 64), jnp.float32),
        ],
    )(s1, xp, w1l, w1r, b1, w2l, w2r, b2)


def _final(s2, s1, r):
    B = 512
    grid = (NP // B,)

    def body(s2_r, s1_r, r_r, out_r):
        inv = 1.0 / jnp.maximum(s1_r[:, 27:28], 1.0)
        out_r[...] = s2_r[...] * inv + r_r[...]

    return pl.pallas_call(
        body,
        grid=grid,
        in_specs=[
            pl.BlockSpec((B, 64), lambda i: (i, 0)),
            pl.BlockSpec((B, 32), lambda i: (i, 0)),
            pl.BlockSpec((B, 64), lambda i: (i, 0)),
        ],
        out_specs=pl.BlockSpec((B, 64), lambda i: (i, 0)),
        out_shape=jax.ShapeDtypeStruct((NP, 64), jnp.float32),
    )(s2, s1, r)


def kernel(x, edge_index, W1_l, W1_r, b1, W2_l, W2_r, b2):
    src = edge_index[0].astype(jnp.int32)
    dst = edge_index[1].astype(jnp.int32)
    srcp = jnp.concatenate(
        [src, jnp.zeros((EPAD - N_EDGES,), jnp.int32)]).reshape(EGROUPS, 128)
    dstp = jnp.concatenate(
        [dst, jnp.full((EPAD - N_EDGES,), 1 << 29, jnp.int32)]).reshape(EGROUPS, 128)

    xp = jnp.concatenate(
        [x, jnp.ones((N_NODES, 1), jnp.float32),
         jnp.zeros((N_NODES, 4), jnp.float32)], axis=1)
    xp = jnp.concatenate(
        [xp, jnp.zeros((NP - N_NODES, 32), jnp.float32)], axis=0)

    S1 = _agg32(xp, srcp, dstp)
    w1l = jnp.pad(W1_l, ((0, 5), (0, 0)))
    w1r = jnp.pad(W1_r, ((0, 5), (0, 0)))
    g, r = _dense_mid(S1[:NP], xp, w1l, w1r,
                      b1.reshape(1, 128), W2_l, W2_r, b2.reshape(1, 64))
    S2 = _agg64(g, srcp, dstp)
    out = _final(S2[:NP], S1[:NP], r)
    return out[:N_NODES]
